# baseline (speedup 1.0000x reference)
import functools

import jax
import jax.numpy as jnp
from jax.experimental import pallas as pl
from jax.experimental.pallas import tpu as pltpu

N_EMBED = 1024
DIM = 64
COMMITMENT_COST = 1.0

ROW_TILE = 512


def _vq_kernel(x_ref, e_ref, q_ref, ind_ref, loss_ref):
    i = pl.program_id(0)
    x = x_ref[...]
    e = e_ref[...]

    xsq = jnp.sum(x * x, axis=1, keepdims=True)
    esq = jnp.sum(e * e, axis=0, keepdims=True)
    xe = jax.lax.dot_general(
        x, e, (((1,), (0,)), ((), ())),
        preferred_element_type=jnp.float32,
    )
    dist = xsq - 2.0 * xe + esq

    idx = jnp.argmin(dist, axis=1).astype(jnp.int32)

    onehot = (jax.lax.broadcasted_iota(jnp.int32, dist.shape, 1)
              == idx[:, None]).astype(jnp.float32)
    q = jax.lax.dot_general(
        onehot, e, (((1,), (1,)), ((), ())),
        preferred_element_type=jnp.float32,
        precision=jax.lax.Precision.HIGHEST,
    )

    diff = q - x
    q_ref[...] = x + diff
    ind_ref[0, 0, :] = idx

    part = jnp.sum(diff * diff)

    @pl.when(i == 0)
    def _():
        loss_ref[0, 0] = part

    @pl.when(i != 0)
    def _():
        loss_ref[0, 0] += part


@functools.partial(jax.jit, static_argnames=())
def kernel(inputs, embed):
    n_total = inputs.shape[0] * inputs.shape[1]
    flatten = inputs.reshape(n_total, DIM)
    grid = n_total // ROW_TILE

    q, ind3, loss_acc = pl.pallas_call(
        _vq_kernel,
        grid=(grid,),
        in_specs=[
            pl.BlockSpec((ROW_TILE, DIM), lambda i: (i, 0)),
            pl.BlockSpec((DIM, N_EMBED), lambda i: (0, 0)),
        ],
        out_specs=[
            pl.BlockSpec((ROW_TILE, DIM), lambda i: (i, 0)),
            pl.BlockSpec((1, 1, ROW_TILE), lambda i: (i, 0, 0)),
            pl.BlockSpec(memory_space=pltpu.SMEM),
        ],
        out_shape=[
            jax.ShapeDtypeStruct((n_total, DIM), jnp.float32),
            jax.ShapeDtypeStruct((grid, 1, ROW_TILE), jnp.int32),
            jax.ShapeDtypeStruct((1, 1), jnp.float32),
        ],
    )(flatten, embed)

    quantize = q.reshape(inputs.shape)
    embed_ind = ind3.reshape(inputs.shape[:-1])
    loss = (loss_acc[0, 0] / jnp.float32(n_total * DIM)) * COMMITMENT_COST
    return (quantize, embed_ind, loss)


# speedup vs baseline: 1.2437x; 1.2437x over previous
import functools

import jax
import jax.numpy as jnp
from jax.experimental import pallas as pl
from jax.experimental.pallas import tpu as pltpu

N_EMBED = 1024
DIM = 64
COMMITMENT_COST = 1.0

ROW_TILE = 512


def _vq_kernel(x_ref, e_ref, ehi_ref, emid_ref, q_ref, ind_ref, loss_ref):
    i = pl.program_id(0)
    x = x_ref[...]
    e = e_ref[...]

    xsq = jnp.sum(x * x, axis=1, keepdims=True)
    esq = jnp.sum(e * e, axis=0, keepdims=True)
    xe = jax.lax.dot_general(
        x, e, (((1,), (0,)), ((), ())),
        preferred_element_type=jnp.float32,
    )
    dist = xsq - 2.0 * xe + esq

    idx = jnp.argmin(dist, axis=1).astype(jnp.int32)

    onehot = (jax.lax.broadcasted_iota(jnp.int32, dist.shape, 1)
              == idx[:, None]).astype(jnp.bfloat16)
    dims = (((1,), (1,)), ((), ()))
    q = (jax.lax.dot_general(onehot, ehi_ref[...], dims,
                             preferred_element_type=jnp.float32)
         + jax.lax.dot_general(onehot, emid_ref[...], dims,
                               preferred_element_type=jnp.float32))

    diff = q - x
    q_ref[...] = x + diff
    ind_ref[0, 0, :] = idx

    part = jnp.sum(diff * diff)

    @pl.when(i == 0)
    def _():
        loss_ref[0, 0] = part

    @pl.when(i != 0)
    def _():
        loss_ref[0, 0] += part


@functools.partial(jax.jit, static_argnames=())
def kernel(inputs, embed):
    n_total = inputs.shape[0] * inputs.shape[1]
    flatten = inputs.reshape(n_total, DIM)
    grid = n_total // ROW_TILE

    e_hi = embed.astype(jnp.bfloat16)
    e_mid = (embed - e_hi.astype(jnp.float32)).astype(jnp.bfloat16)

    q, ind3, loss_acc = pl.pallas_call(
        _vq_kernel,
        grid=(grid,),
        in_specs=[
            pl.BlockSpec((ROW_TILE, DIM), lambda i: (i, 0)),
            pl.BlockSpec((DIM, N_EMBED), lambda i: (0, 0)),
            pl.BlockSpec((DIM, N_EMBED), lambda i: (0, 0)),
            pl.BlockSpec((DIM, N_EMBED), lambda i: (0, 0)),
        ],
        out_specs=[
            pl.BlockSpec((ROW_TILE, DIM), lambda i: (i, 0)),
            pl.BlockSpec((1, 1, ROW_TILE), lambda i: (i, 0, 0)),
            pl.BlockSpec(memory_space=pltpu.SMEM),
        ],
        out_shape=[
            jax.ShapeDtypeStruct((n_total, DIM), jnp.float32),
            jax.ShapeDtypeStruct((grid, 1, ROW_TILE), jnp.int32),
            jax.ShapeDtypeStruct((1, 1), jnp.float32),
        ],
    )(flatten, embed, e_hi, e_mid)

    quantize = q.reshape(inputs.shape)
    embed_ind = ind3.reshape(inputs.shape[:-1])
    loss = (loss_acc[0, 0] / jnp.float32(n_total * DIM)) * COMMITMENT_COST
    return (quantize, embed_ind, loss)


# speedup vs baseline: 1.6874x; 1.3568x over previous
import functools

import jax
import jax.numpy as jnp
from jax.experimental import pallas as pl
from jax.experimental.pallas import tpu as pltpu

N_EMBED = 1024
DIM = 64
COMMITMENT_COST = 1.0

ROW_TILE = 1152


def _vq_kernel(x_ref, e_ref, ehi_ref, q_ref, ind_ref, loss_ref):
    i = pl.program_id(0)
    x = x_ref[...]
    e = e_ref[...]

    xsq = jnp.sum(x * x, axis=1, keepdims=True)
    esq = jnp.sum(e * e, axis=0, keepdims=True)
    xe = jax.lax.dot_general(
        x, e, (((1,), (0,)), ((), ())),
        preferred_element_type=jnp.float32,
    )
    dist = xsq - 2.0 * xe + esq

    idx = jnp.argmin(dist, axis=1).astype(jnp.int32)

    onehot = (jax.lax.broadcasted_iota(jnp.int32, dist.shape, 1)
              == idx[:, None]).astype(jnp.bfloat16)
    q = jax.lax.dot_general(
        onehot, ehi_ref[...], (((1,), (1,)), ((), ())),
        preferred_element_type=jnp.float32)

    diff = q - x
    q_ref[...] = x + diff
    ind_ref[0, 0, :] = idx

    part = jnp.sum(diff * diff)

    @pl.when(i == 0)
    def _():
        loss_ref[0, 0] = part

    @pl.when(i != 0)
    def _():
        loss_ref[0, 0] += part


@functools.partial(jax.jit, static_argnames=())
def kernel(inputs, embed):
    n_total = inputs.shape[0] * inputs.shape[1]
    flatten = inputs.reshape(n_total, DIM)
    grid = n_total // ROW_TILE

    e_hi = embed.astype(jnp.bfloat16)

    q, ind3, loss_acc = pl.pallas_call(
        _vq_kernel,
        grid=(grid,),
        in_specs=[
            pl.BlockSpec((ROW_TILE, DIM), lambda i: (i, 0)),
            pl.BlockSpec((DIM, N_EMBED), lambda i: (0, 0)),
            pl.BlockSpec((DIM, N_EMBED), lambda i: (0, 0)),
        ],
        out_specs=[
            pl.BlockSpec((ROW_TILE, DIM), lambda i: (i, 0)),
            pl.BlockSpec((1, 1, ROW_TILE), lambda i: (i, 0, 0)),
            pl.BlockSpec(memory_space=pltpu.SMEM),
        ],
        out_shape=[
            jax.ShapeDtypeStruct((n_total, DIM), jnp.float32),
            jax.ShapeDtypeStruct((grid, 1, ROW_TILE), jnp.int32),
            jax.ShapeDtypeStruct((1, 1), jnp.float32),
        ],
    )(flatten, embed, e_hi)

    quantize = q.reshape(inputs.shape)
    embed_ind = ind3.reshape(inputs.shape[:-1])
    loss = (loss_acc[0, 0] / jnp.float32(n_total * DIM)) * COMMITMENT_COST
    return (quantize, embed_ind, loss)


# speedup vs baseline: 1.7331x; 1.0271x over previous
import functools

import jax
import jax.numpy as jnp
from jax.experimental import pallas as pl
from jax.experimental.pallas import tpu as pltpu

N_EMBED = 1024
DIM = 64
COMMITMENT_COST = 1.0

ROW_TILE = 2304


def _vq_kernel(x_ref, e_ref, ehi_ref, q_ref, ind_ref, loss_ref):
    i = pl.program_id(0)
    x = x_ref[...]
    e = e_ref[...]

    xsq = jnp.sum(x * x, axis=1, keepdims=True)
    esq = jnp.sum(e * e, axis=0, keepdims=True)
    xe = jax.lax.dot_general(
        x, e, (((1,), (0,)), ((), ())),
        preferred_element_type=jnp.float32,
    )
    dist = xsq - 2.0 * xe + esq

    idx = jnp.argmin(dist, axis=1).astype(jnp.int32)

    onehot = (jax.lax.broadcasted_iota(jnp.int32, dist.shape, 1)
              == idx[:, None]).astype(jnp.bfloat16)
    q = jax.lax.dot_general(
        onehot, ehi_ref[...], (((1,), (1,)), ((), ())),
        preferred_element_type=jnp.float32)

    diff = q - x
    q_ref[...] = x + diff
    ind_ref[0, 0, :] = idx

    part = jnp.sum(diff * diff)

    @pl.when(i == 0)
    def _():
        loss_ref[0, 0] = part

    @pl.when(i != 0)
    def _():
        loss_ref[0, 0] += part


@functools.partial(jax.jit, static_argnames=())
def kernel(inputs, embed):
    n_total = inputs.shape[0] * inputs.shape[1]
    flatten = inputs.reshape(n_total, DIM)
    grid = n_total // ROW_TILE

    e_hi = embed.astype(jnp.bfloat16)

    q, ind3, loss_acc = pl.pallas_call(
        _vq_kernel,
        grid=(grid,),
        in_specs=[
            pl.BlockSpec((ROW_TILE, DIM), lambda i: (i, 0)),
            pl.BlockSpec((DIM, N_EMBED), lambda i: (0, 0)),
            pl.BlockSpec((DIM, N_EMBED), lambda i: (0, 0)),
        ],
        out_specs=[
            pl.BlockSpec((ROW_TILE, DIM), lambda i: (i, 0)),
            pl.BlockSpec((1, 1, ROW_TILE), lambda i: (i, 0, 0)),
            pl.BlockSpec(memory_space=pltpu.SMEM),
        ],
        out_shape=[
            jax.ShapeDtypeStruct((n_total, DIM), jnp.float32),
            jax.ShapeDtypeStruct((grid, 1, ROW_TILE), jnp.int32),
            jax.ShapeDtypeStruct((1, 1), jnp.float32),
        ],
    )(flatten, embed, e_hi)

    quantize = q.reshape(inputs.shape)
    embed_ind = ind3.reshape(inputs.shape[:-1])
    loss = (loss_acc[0, 0] / jnp.float32(n_total * DIM)) * COMMITMENT_COST
    return (quantize, embed_ind, loss)
